# baseline (speedup 1.0000x reference)
import functools

import jax
import jax.numpy as jnp
from jax import lax
from jax.experimental import pallas as pl
from jax.experimental.pallas import tpu as pltpu
from jax.experimental.pallas import tpu_sc as plsc

_NUM_CORES = 2
_NUM_SUBCORES = 16
_NUM_WORKERS = _NUM_CORES * _NUM_SUBCORES
_LANES = 16

_NUM_EVENTS = 1969
_BATCH = 16384
_D_OUT = 5
_D_PAD = 8
_B_PER_W = _BATCH // _NUM_WORKERS


def _project_body(ev_ref, w_ref, b_ref, out_ref):
    t = (
        jnp.dot(ev_ref[...], w_ref[...], preferred_element_type=jnp.float32)
        + b_ref[...]
    )
    out_ref[...] = jnp.pad(t, ((0, 0), (0, _D_PAD - _D_OUT)))


def _project(events, w, b2d):
    return pl.pallas_call(
        _project_body,
        out_shape=jax.ShapeDtypeStruct((_NUM_EVENTS, _D_PAD), jnp.float32),
    )(events, w, b2d)


_sc_mesh = plsc.VectorSubcoreMesh(
    core_axis_name="c",
    subcore_axis_name="s",
    num_cores=_NUM_CORES,
    num_subcores=_NUM_SUBCORES,
)


@functools.partial(
    pl.kernel,
    out_type=jax.ShapeDtypeStruct((_D_OUT * _BATCH,), jnp.float32),
    mesh=_sc_mesh,
    scratch_types=[
        pltpu.VMEM((_B_PER_W,), jnp.int32),
        pltpu.VMEM((_B_PER_W, _D_PAD), jnp.float32),
        [pltpu.VMEM((_B_PER_W,), jnp.float32) for _ in range(_D_OUT)],
        pltpu.SemaphoreType.DMA,
    ],
    compiler_params=pltpu.CompilerParams(
        use_tc_tiling_on_sc=False, needs_layout_passes=False
    ),
)
def _gather_rows(table_hbm, idx_hbm, out_hbm, idx_v, rows_v, tcol, sem):
    wid = lax.axis_index("s") * _NUM_CORES + lax.axis_index("c")
    base = wid * _B_PER_W
    pltpu.sync_copy(idx_hbm.at[pl.ds(base, _B_PER_W)], idx_v)
    pltpu.async_copy(table_hbm.at[idx_v], rows_v, sem).wait()
    lanes = lax.iota(jnp.int32, _LANES)
    csplat = [lanes * 0 + c for c in range(_D_OUT)]

    @plsc.parallel_loop(0, _B_PER_W, step=_LANES)
    def _transpose_block(i):
        rows = lanes + i
        for c in range(_D_OUT):
            tcol[c][pl.ds(i, _LANES)] = plsc.load_gather(
                rows_v, [rows, csplat[c]]
            )

    for c in range(_D_OUT):
        pltpu.sync_copy(tcol[c], out_hbm.at[pl.ds(c * _BATCH + base, _B_PER_W)])


def kernel(days_index, events, W, b):
    table = _project(events, W, b.reshape(1, _D_OUT))
    flat = _gather_rows(table, days_index)
    return flat.reshape(_D_OUT, _BATCH).T


# speedup vs baseline: 1.0364x; 1.0364x over previous
import functools

import jax
import jax.numpy as jnp
from jax import lax
from jax.experimental import pallas as pl
from jax.experimental.pallas import tpu as pltpu
from jax.experimental.pallas import tpu_sc as plsc

_NUM_CORES = 1
_NUM_SUBCORES = 16
_NUM_WORKERS = _NUM_CORES * _NUM_SUBCORES
_LANES = 16

_NUM_EVENTS = 1969
_BATCH = 16384
_D_OUT = 5
_D_PAD = 8
_B_PER_W = _BATCH // _NUM_WORKERS


def _project_body(ev_ref, w_ref, b_ref, out_ref):
    t = (
        jnp.dot(ev_ref[...], w_ref[...], preferred_element_type=jnp.float32)
        + b_ref[...]
    )
    out_ref[...] = jnp.pad(t, ((0, 0), (0, _D_PAD - _D_OUT)))


def _project(events, w, b2d):
    return pl.pallas_call(
        _project_body,
        out_shape=jax.ShapeDtypeStruct((_NUM_EVENTS, _D_PAD), jnp.float32),
    )(events, w, b2d)


_sc_mesh = plsc.VectorSubcoreMesh(
    core_axis_name="c",
    subcore_axis_name="s",
    num_cores=_NUM_CORES,
    num_subcores=_NUM_SUBCORES,
)


@functools.partial(
    pl.kernel,
    out_type=jax.ShapeDtypeStruct((_D_OUT * _BATCH,), jnp.float32),
    mesh=_sc_mesh,
    scratch_types=[
        pltpu.VMEM((_B_PER_W,), jnp.int32),
        pltpu.VMEM((_B_PER_W, _D_PAD), jnp.float32),
        [pltpu.VMEM((_B_PER_W,), jnp.float32) for _ in range(_D_OUT)],
        pltpu.SemaphoreType.DMA,
    ],
    compiler_params=pltpu.CompilerParams(
        use_tc_tiling_on_sc=False, needs_layout_passes=False
    ),
)
def _gather_rows(table_hbm, idx_hbm, out_hbm, idx_v, rows_v, tcol, sem):
    wid = lax.axis_index("s") * _NUM_CORES + lax.axis_index("c")
    base = wid * _B_PER_W
    pltpu.sync_copy(idx_hbm.at[pl.ds(base, _B_PER_W)], idx_v)
    pltpu.async_copy(table_hbm.at[idx_v], rows_v, sem).wait()
    lanes = lax.iota(jnp.int32, _LANES)
    csplat = [lanes * 0 + c for c in range(_D_OUT)]

    @plsc.parallel_loop(0, _B_PER_W, step=_LANES)
    def _transpose_block(i):
        rows = lanes + i
        for c in range(_D_OUT):
            tcol[c][pl.ds(i, _LANES)] = plsc.load_gather(
                rows_v, [rows, csplat[c]]
            )

    handles = [
        pltpu.async_copy(
            tcol[c], out_hbm.at[pl.ds(c * _BATCH + base, _B_PER_W)], sem
        )
        for c in range(_D_OUT)
    ]
    for h in handles:
        h.wait()


def kernel(days_index, events, W, b):
    table = _project(events, W, b.reshape(1, _D_OUT))
    flat = _gather_rows(table, days_index)
    return flat.reshape(_D_OUT, _BATCH).T
